# baseline (speedup 1.0000x reference)
import jax
import jax.numpy as jnp
from jax import lax
from jax.experimental import pallas as pl
from jax.experimental.pallas import tpu as pltpu
from jax.experimental.pallas import tpu_sc as plsc

NC = 2
NS = 16
L = 16
NW = NC * NS

BATCH = 16384
DIM = 32
BPW = BATCH // NW
GROUPS = BPW // L


def _rsqrt_clip(x):
    i = plsc.bitcast(x, jnp.int32)
    y = plsc.bitcast(jnp.int32(0x5F3759DF) - (i >> 1), jnp.float32)
    for _ in range(3):
        y = y * (1.5 - 0.5 * x * y * y)
    return jnp.minimum(jnp.float32(1.0), y)


def _body(users_hbm, pos_hbm, neg_hbm, ut_hbm, it_hbm, out_hbm,
          idx_u, idx_p, idx_n, rows_u, rows_p, rows_n, out_v,
          sem_u, sem_p, sem_n):
    wid = lax.axis_index("s") * NC + lax.axis_index("c")
    base = wid * BPW

    pltpu.sync_copy(users_hbm.at[pl.ds(base, BPW)], idx_u)
    pltpu.sync_copy(pos_hbm.at[pl.ds(base, BPW)], idx_p)
    pltpu.sync_copy(neg_hbm.at[pl.ds(base, BPW)], idx_n)

    cu = pltpu.async_copy(ut_hbm.at[idx_u], rows_u, sem_u)
    cp = pltpu.async_copy(it_hbm.at[idx_p], rows_p, sem_p)
    cn = pltpu.async_copy(it_hbm.at[idx_n], rows_n, sem_n)
    cu.wait()
    cp.wait()
    cn.wait()

    lane = lax.iota(jnp.int32, L)

    @pl.loop(0, GROUPS)
    def _group(g):
        rid = g * L + lane
        zero = jnp.zeros((L,), jnp.float32)
        nu = zero
        npp = zero
        nn = zero
        dp = zero
        dn = zero
        for d in range(DIM):
            col = jnp.full((L,), d, jnp.int32)
            u = plsc.load_gather(rows_u, [rid, col])
            p = plsc.load_gather(rows_p, [rid, col])
            n = plsc.load_gather(rows_n, [rid, col])
            nu = nu + u * u
            npp = npp + p * p
            nn = nn + n * n
            dp = dp + u * p
            dn = dn + u * n
        su = _rsqrt_clip(nu)
        sp = _rsqrt_clip(npp)
        sn = _rsqrt_clip(nn)
        out_v[pl.ds(g * L, L)] = su * (dp * sp - dn * sn)

    pltpu.sync_copy(out_v, out_hbm.at[pl.ds(base, BPW)])


def kernel(users, pos_items, neg_items, user_table, item_table):
    mesh = plsc.VectorSubcoreMesh(
        core_axis_name="c", subcore_axis_name="s",
        num_cores=NC, num_subcores=NS)
    f = pl.kernel(
        _body,
        out_type=jax.ShapeDtypeStruct((BATCH,), jnp.float32),
        mesh=mesh,
        compiler_params=pltpu.CompilerParams(
            needs_layout_passes=False, use_tc_tiling_on_sc=False),
        scratch_types=[
            pltpu.VMEM((BPW,), jnp.int32),
            pltpu.VMEM((BPW,), jnp.int32),
            pltpu.VMEM((BPW,), jnp.int32),
            pltpu.VMEM((BPW, DIM), jnp.float32),
            pltpu.VMEM((BPW, DIM), jnp.float32),
            pltpu.VMEM((BPW, DIM), jnp.float32),
            pltpu.VMEM((BPW,), jnp.float32),
            pltpu.SemaphoreType.DMA,
            pltpu.SemaphoreType.DMA,
            pltpu.SemaphoreType.DMA,
        ],
    )
    return f(users, pos_items, neg_items, user_table, item_table)
